# baseline (speedup 1.0000x reference)
import functools

import jax
import jax.numpy as jnp
from jax import lax
from jax.experimental import pallas as pl
from jax.experimental.pallas import tpu as pltpu
from jax.experimental.pallas import tpu_sc as plsc


def _make_gather2(B, D2):
    try:
        info = plsc.get_sparse_core_info()
        NC, NS = info.num_cores, info.num_subcores
    except Exception:
        NC, NS = 2, 16
    NW = NC * NS
    assert B % (8 * NW) == 0
    b_per_w = B // NW

    mesh = plsc.VectorSubcoreMesh(core_axis_name="c", subcore_axis_name="s")

    @functools.partial(
        pl.kernel,
        mesh=mesh,
        out_type=[
            jax.ShapeDtypeStruct((B, D2), jnp.float32),
            jax.ShapeDtypeStruct((B, D2), jnp.float32),
        ],
        scratch_types=[
            pltpu.VMEM((b_per_w,), jnp.int32),
            pltpu.VMEM((b_per_w, D2), jnp.float32),
            pltpu.SemaphoreType.DMA,
        ],
    )
    def gather2(ublk_hbm, iblk_hbm, u_tab_hbm, v_tab_hbm, u_out_hbm,
                v_out_hbm, idx, rows, sem):
        wid = lax.axis_index("s") * NC + lax.axis_index("c")
        base = wid * b_per_w
        for idx_hbm, tab_hbm, out_hbm in (
            (ublk_hbm, u_tab_hbm, u_out_hbm),
            (iblk_hbm, v_tab_hbm, v_out_hbm),
        ):
            pltpu.sync_copy(idx_hbm.at[pl.ds(base, b_per_w)], idx)
            pltpu.async_copy(tab_hbm.at[idx], rows, sem).wait()
            pltpu.sync_copy(rows, out_hbm.at[pl.ds(base, b_per_w)])

    return gather2


def kernel(users, items, U_e, V_e):
    B = users.shape[0]
    D = U_e.shape[1]
    U2 = U_e.reshape(-1, 2 * D)
    V2 = V_e.reshape(-1, 2 * D)
    ui = users.astype(jnp.int32)
    ii = items.astype(jnp.int32)
    fn = _make_gather2(B, 2 * D)
    u2, v2 = fn(ui >> 1, ii >> 1, U2, V2)
    u_e = jnp.where((ui & 1).astype(bool)[:, None], u2[:, D:], u2[:, :D])
    v_e = jnp.where((ii & 1).astype(bool)[:, None], v2[:, D:], v2[:, :D])
    return (u_e, v_e)


# speedup vs baseline: 2.3880x; 2.3880x over previous
import functools

import jax
import jax.numpy as jnp
from jax import lax
from jax.experimental import pallas as pl
from jax.experimental.pallas import tpu as pltpu
from jax.experimental.pallas import tpu_sc as plsc

_NBUF = 4
_LANES = 128
_G = 16


def _make_gather2(B, D):
    try:
        info = plsc.get_sparse_core_info()
        NC, NS = info.num_cores, info.num_subcores
    except Exception:
        NC, NS = 2, 16
    NW = NC * NS
    assert B % (8 * NW) == 0
    b_per_w = B // NW
    n_groups = b_per_w // _G

    mesh = plsc.VectorSubcoreMesh(core_axis_name="c", subcore_axis_name="s")

    @functools.partial(
        pl.kernel,
        mesh=mesh,
        out_type=[
            jax.ShapeDtypeStruct((B, D), jnp.float32),
            jax.ShapeDtypeStruct((B, D), jnp.float32),
        ],
        scratch_types=[
            pltpu.VMEM((b_per_w,), jnp.int32),
            pltpu.VMEM((_NBUF, D, _LANES), jnp.float32),
            pltpu.VMEM((b_per_w, D), jnp.float32),
        ]
        + [pltpu.SemaphoreType.DMA] * _NBUF,
        compiler_params=pltpu.CompilerParams(needs_layout_passes=False),
    )
    def gather2(uidx_hbm, iidx_hbm, u_tab_hbm, v_tab_hbm, u_out_hbm,
                v_out_hbm, idx_v, blk, rowbuf, *sems):
        wid = lax.axis_index("s") * NC + lax.axis_index("c")
        base = wid * b_per_w
        lanes = lax.iota(jnp.int32, _G)
        rows16 = [lax.iota(jnp.int32, 16) + 16 * g for g in range(D // 16)]

        def extract(vec, l):
            return jnp.sum(jnp.where(lanes == l, vec, 0))

        for idx_hbm, tab_hbm, out_hbm in (
            (uidx_hbm, u_tab_hbm, u_out_hbm),
            (iidx_hbm, v_tab_hbm, v_out_hbm),
        ):
            pltpu.sync_copy(idx_hbm.at[pl.ds(base, b_per_w)], idx_v)

            def issue(c, b):
                cb = pl.multiple_of((c >> 7) << 7, _LANES)
                pltpu.async_copy(tab_hbm.at[:, pl.ds(cb, _LANES)],
                                 blk.at[b], sems[b])

            vec0 = idx_v[pl.ds(0, _G)]
            for b in range(_NBUF):
                issue(extract(vec0, b), b)

            def group_body(g, carry):
                vec = idx_v[pl.ds(g * _G, _G)]
                for l in range(_G):
                    r = g * _G + l
                    b = l % _NBUF
                    pltpu.make_async_copy(
                        tab_hbm.at[:, pl.ds(0, _LANES)], blk.at[b],
                        sems[b]).wait()
                    cm = extract(vec, l) & (_LANES - 1)
                    cols = jnp.zeros((16,), jnp.int32) + cm
                    for gg in range(D // 16):
                        v = plsc.load_gather(blk.at[b], [rows16[gg], cols])
                        rowbuf[r, pl.ds(16 * gg, 16)] = v
                    nr = r + _NBUF

                    @pl.when(nr < b_per_w)
                    def _():
                        if l < _G - _NBUF:
                            c_next = extract(vec, l + _NBUF)
                        else:
                            vec_next = idx_v[pl.ds((g + 1) * _G, _G)]
                            c_next = extract(vec_next, l + _NBUF - _G)
                        issue(c_next, b)

                return carry

            lax.fori_loop(0, n_groups, group_body, 0)
            pltpu.sync_copy(rowbuf, out_hbm.at[pl.ds(base, b_per_w)])

    return gather2


def kernel(users, items, U_e, V_e):
    B = users.shape[0]
    D = U_e.shape[1]
    fn = _make_gather2(B, D)
    u_e, v_e = fn(users.astype(jnp.int32), items.astype(jnp.int32),
                  jnp.swapaxes(U_e, 0, 1), jnp.swapaxes(V_e, 0, 1))
    return (u_e, v_e)


# speedup vs baseline: 2.8768x; 1.2047x over previous
import functools

import jax
import jax.numpy as jnp
from jax import lax
from jax.experimental import pallas as pl
from jax.experimental.pallas import tpu as pltpu
from jax.experimental.pallas import tpu_sc as plsc

_NBUF = 8
_LANES = 128
_G = 16


def _make_gather2(B, D):
    try:
        info = plsc.get_sparse_core_info()
        NC, NS = info.num_cores, info.num_subcores
    except Exception:
        NC, NS = 2, 16
    NW = NC * NS
    assert B % (8 * NW) == 0
    b_per_w = B // NW
    n_groups = b_per_w // _G

    mesh = plsc.VectorSubcoreMesh(core_axis_name="c", subcore_axis_name="s")

    @functools.partial(
        pl.kernel,
        mesh=mesh,
        out_type=[
            jax.ShapeDtypeStruct((D, B), jnp.float32),
            jax.ShapeDtypeStruct((D, B), jnp.float32),
        ],
        scratch_types=[
            pltpu.VMEM((b_per_w,), jnp.int32),
            pltpu.VMEM((_NBUF, D, _LANES), jnp.float32),
            pltpu.VMEM((D, b_per_w), jnp.float32),
        ]
        + [pltpu.SemaphoreType.DMA] * _NBUF,
        compiler_params=pltpu.CompilerParams(needs_layout_passes=False),
    )
    def gather2(uidx_hbm, iidx_hbm, u_tab_hbm, v_tab_hbm, u_out_hbm,
                v_out_hbm, idx_v, blk, tbuf, *sems):
        wid = lax.axis_index("s") * NC + lax.axis_index("c")
        base = wid * b_per_w
        lanes = lax.iota(jnp.int32, _G)
        rows16 = [lax.iota(jnp.int32, 16) + 16 * g for g in range(D // 16)]

        def extract(vec, l):
            return jnp.sum(jnp.where(lanes == l, vec, 0))

        for idx_hbm, tab_hbm, out_hbm in (
            (uidx_hbm, u_tab_hbm, u_out_hbm),
            (iidx_hbm, v_tab_hbm, v_out_hbm),
        ):
            pltpu.sync_copy(idx_hbm.at[pl.ds(base, b_per_w)], idx_v)

            def issue(c, b):
                cb = pl.multiple_of((c >> 7) << 7, _LANES)
                pltpu.async_copy(tab_hbm.at[:, pl.ds(cb, _LANES)],
                                 blk.at[b], sems[b])

            vec0 = idx_v[pl.ds(0, _G)]
            for b in range(_NBUF):
                issue(extract(vec0, b), b)

            def group_body(g, carry):
                vec = idx_v[pl.ds(g * _G, _G)]
                for l in range(_G):
                    r = g * _G + l
                    b = l % _NBUF
                    pltpu.make_async_copy(
                        tab_hbm.at[:, pl.ds(0, _LANES)], blk.at[b],
                        sems[b]).wait()
                    cm = extract(vec, l) & (_LANES - 1)
                    cols = jnp.zeros((16,), jnp.int32) + cm
                    rcol = jnp.zeros((16,), jnp.int32) + r
                    for gg in range(D // 16):
                        v = plsc.load_gather(blk.at[b], [rows16[gg], cols])
                        plsc.store_scatter(tbuf, [rows16[gg], rcol], v)
                    nr = r + _NBUF

                    @pl.when(nr < b_per_w)
                    def _():
                        if l < _G - _NBUF:
                            c_next = extract(vec, l + _NBUF)
                        else:
                            vec_next = idx_v[pl.ds((g + 1) * _G, _G)]
                            c_next = extract(vec_next, l + _NBUF - _G)
                        issue(c_next, b)

                return carry

            lax.fori_loop(0, n_groups, group_body, 0)
            pltpu.sync_copy(tbuf, out_hbm.at[:, pl.ds(base, b_per_w)])

    return gather2


def kernel(users, items, U_e, V_e):
    B = users.shape[0]
    D = U_e.shape[1]
    fn = _make_gather2(B, D)
    u_t, v_t = fn(users.astype(jnp.int32), items.astype(jnp.int32),
                  jnp.swapaxes(U_e, 0, 1), jnp.swapaxes(V_e, 0, 1))
    return (jnp.swapaxes(u_t, 0, 1), jnp.swapaxes(v_t, 0, 1))
